# Initial kernel scaffold; baseline (speedup 1.0000x reference)
import jax
import jax.numpy as jnp
from jax.experimental import pallas as pl


def kernel(embeds, centroids, r):
    raise NotImplementedError("write your pallas kernel here")


# speedup vs baseline: 35.8390x; 35.8390x over previous
import jax
import jax.numpy as jnp
from jax.experimental import pallas as pl
from jax.experimental.pallas import tpu as pltpu

_B = 8
_N = 4096
_D = 512
_C = 1024
_NU = 0.001
_K = 1
_TB = 512
_NBLK = (_B * _N) // _TB
_LOSS_SCALE = 1.0 / (_NU * _B * _N * _K)


def _dist_kernel(e_ref, ct_ref, r_ref, score_ref, loss_ref):
    i = pl.program_id(0)
    e = e_ref[...]
    ct = ct_ref[...]
    p = jnp.dot(e, ct, preferred_element_type=jnp.float32)
    centers = jnp.sum(ct * ct, axis=0, keepdims=True)
    d2 = centers - 2.0 * p
    m = jnp.min(d2, axis=1, keepdims=True)
    feat = jnp.sum(e * e, axis=1, keepdims=True)
    dist = jnp.sqrt(feat + m)
    score_ref[...] = dist
    part = jnp.sum(jnp.maximum(dist - r_ref[0] * r_ref[0], 0.0))

    @pl.when(i == 0)
    def _init():
        loss_ref[0, 0] = 0.0

    loss_ref[0, 0] += part

    @pl.when(i == _NBLK - 1)
    def _finish():
        loss_ref[0, 0] = loss_ref[0, 0] * _LOSS_SCALE


def kernel(embeds, centroids, r):
    e2d = embeds.reshape(_B * _N, _D)
    ct = centroids.T
    score_flat, loss = pl.pallas_call(
        _dist_kernel,
        grid=(_NBLK,),
        in_specs=[
            pl.BlockSpec((_TB, _D), lambda i: (i, 0)),
            pl.BlockSpec((_D, _C), lambda i: (0, 0)),
            pl.BlockSpec(memory_space=pltpu.SMEM),
        ],
        out_specs=[
            pl.BlockSpec((_TB, 1), lambda i: (i, 0)),
            pl.BlockSpec(memory_space=pltpu.SMEM),
        ],
        out_shape=[
            jax.ShapeDtypeStruct((_B * _N, 1), jnp.float32),
            jax.ShapeDtypeStruct((1, 1), jnp.float32),
        ],
        compiler_params=pltpu.CompilerParams(
            dimension_semantics=("arbitrary",),
        ),
    )(e2d, ct, r)
    h = 64
    score = score_flat.reshape(_B, 1, h, h)
    return (loss[0, 0], score, embeds)


# speedup vs baseline: 36.2021x; 1.0101x over previous
import jax
import jax.numpy as jnp
from jax.experimental import pallas as pl
from jax.experimental.pallas import tpu as pltpu

_B = 8
_N = 4096
_D = 512
_C = 1024
_NU = 0.001
_K = 1
_TB = 512
_NBLK = (_B * _N) // _TB
_LOSS_SCALE = 1.0 / (_NU * _B * _N * _K)


def _dist_kernel(e_ref, ct_ref, r_ref, score_ref, loss_ref, ctm2_ref, cen_ref):
    i = pl.program_id(0)

    @pl.when(i == 0)
    def _prep():
        ct = ct_ref[...]
        ctm2_ref[...] = (-2.0 * ct).astype(jnp.bfloat16)
        cen_ref[...] = jnp.sum(ct * ct, axis=0, keepdims=True)

    e = e_ref[...]
    p = jnp.dot(e.astype(jnp.bfloat16), ctm2_ref[...],
                preferred_element_type=jnp.float32)
    d2 = cen_ref[...] + p
    m = jnp.min(d2, axis=1, keepdims=True)
    feat = jnp.sum(e * e, axis=1, keepdims=True)
    dist = jnp.sqrt(feat + m)
    score_ref[...] = dist
    part = jnp.sum(jnp.maximum(dist - r_ref[0] * r_ref[0], 0.0))

    @pl.when(i == 0)
    def _init():
        loss_ref[0, 0] = 0.0

    loss_ref[0, 0] += part

    @pl.when(i == _NBLK - 1)
    def _finish():
        loss_ref[0, 0] = loss_ref[0, 0] * _LOSS_SCALE


def kernel(embeds, centroids, r):
    e2d = embeds.reshape(_B * _N, _D)
    ct = centroids.T
    score_flat, loss = pl.pallas_call(
        _dist_kernel,
        grid=(_NBLK,),
        in_specs=[
            pl.BlockSpec((_TB, _D), lambda i: (i, 0)),
            pl.BlockSpec((_D, _C), lambda i: (0, 0)),
            pl.BlockSpec(memory_space=pltpu.SMEM),
        ],
        out_specs=[
            pl.BlockSpec((_TB, 1), lambda i: (i, 0)),
            pl.BlockSpec(memory_space=pltpu.SMEM),
        ],
        out_shape=[
            jax.ShapeDtypeStruct((_B * _N, 1), jnp.float32),
            jax.ShapeDtypeStruct((1, 1), jnp.float32),
        ],
        scratch_shapes=[
            pltpu.VMEM((_D, _C), jnp.bfloat16),
            pltpu.VMEM((1, _C), jnp.float32),
        ],
        compiler_params=pltpu.CompilerParams(
            dimension_semantics=("arbitrary",),
        ),
    )(e2d, ct, r)
    h = 64
    score = score_flat.reshape(_B, 1, h, h)
    return (loss[0, 0], score, embeds)


# speedup vs baseline: 47.1724x; 1.3030x over previous
import jax
import jax.numpy as jnp
from jax.experimental import pallas as pl
from jax.experimental.pallas import tpu as pltpu

_B = 8
_N = 4096
_D = 512
_C = 1024
_NU = 0.001
_K = 1
_TB = 512
_NBLK = (_B * _N) // _TB
_LOSS_SCALE = 1.0 / (_NU * _B * _N * _K)


def _dist_kernel(e_ref, ct_ref, r_ref, score_ref, loss_ref, eout_ref,
                 ctm2_ref, cen_ref):
    i = pl.program_id(0)

    @pl.when(i == 0)
    def _prep():
        ct = ct_ref[...]
        ctm2_ref[...] = (-2.0 * ct).astype(jnp.bfloat16)
        cen_ref[...] = jnp.sum(ct * ct, axis=0, keepdims=True)

    e = e_ref[...]
    p = jnp.dot(e.astype(jnp.bfloat16), ctm2_ref[...],
                preferred_element_type=jnp.float32)
    d2 = cen_ref[...] + p
    m = jnp.min(d2, axis=1, keepdims=True)
    feat = jnp.sum(e * e, axis=1, keepdims=True)
    dist = jnp.sqrt(feat + m)
    score_ref[...] = dist
    eout_ref[...] = e
    part = jnp.sum(jnp.maximum(dist - r_ref[0] * r_ref[0], 0.0))

    @pl.when(i == 0)
    def _init():
        loss_ref[0, 0] = 0.0

    loss_ref[0, 0] += part

    @pl.when(i == _NBLK - 1)
    def _finish():
        loss_ref[0, 0] = loss_ref[0, 0] * _LOSS_SCALE


def kernel(embeds, centroids, r):
    e2d = embeds.reshape(_B * _N, _D)
    ct = centroids.T
    score_flat, loss, e_out = pl.pallas_call(
        _dist_kernel,
        grid=(_NBLK,),
        in_specs=[
            pl.BlockSpec((_TB, _D), lambda i: (i, 0)),
            pl.BlockSpec((_D, _C), lambda i: (0, 0)),
            pl.BlockSpec(memory_space=pltpu.SMEM),
        ],
        out_specs=[
            pl.BlockSpec((_TB, 1), lambda i: (i, 0)),
            pl.BlockSpec(memory_space=pltpu.SMEM),
            pl.BlockSpec((_TB, _D), lambda i: (i, 0)),
        ],
        out_shape=[
            jax.ShapeDtypeStruct((_B * _N, 1), jnp.float32),
            jax.ShapeDtypeStruct((1, 1), jnp.float32),
            jax.ShapeDtypeStruct((_B * _N, _D), jnp.float32),
        ],
        scratch_shapes=[
            pltpu.VMEM((_D, _C), jnp.bfloat16),
            pltpu.VMEM((1, _C), jnp.float32),
        ],
        compiler_params=pltpu.CompilerParams(
            dimension_semantics=("arbitrary",),
        ),
    )(e2d, ct, r)
    h = 64
    score = score_flat.reshape(_B, 1, h, h)
    return (loss[0, 0], score, e_out.reshape(_B, _N, _D))


# speedup vs baseline: 59.7333x; 1.2663x over previous
import jax
import jax.numpy as jnp
from jax.experimental import pallas as pl
from jax.experimental.pallas import tpu as pltpu

_B = 8
_N = 4096
_D = 512
_C = 1024
_NU = 0.001
_K = 1
_TB = 1024
_NBLK = (_B * _N) // _TB
_LOSS_SCALE = 1.0 / (_NU * _B * _N * _K)


def _dist_kernel(e_ref, ct_ref, r_ref, score_ref, loss_ref, eout_ref,
                 ctm2_ref, cen_ref):
    i = pl.program_id(0)

    @pl.when(i == 0)
    def _prep():
        ct = ct_ref[...]
        ctm2_ref[...] = (-2.0 * ct).astype(jnp.bfloat16)
        cen_ref[...] = jnp.sum(ct * ct, axis=0, keepdims=True)

    e = e_ref[...]
    p = jnp.dot(e.astype(jnp.bfloat16), ctm2_ref[...],
                preferred_element_type=jnp.float32)
    d2 = cen_ref[...] + p
    m = jnp.min(d2, axis=1, keepdims=True)
    feat = jnp.sum(e * e, axis=1, keepdims=True)
    dist = jnp.sqrt(feat + m)
    score_ref[...] = dist
    eout_ref[...] = e
    part = jnp.sum(jnp.maximum(dist - r_ref[0] * r_ref[0], 0.0))

    @pl.when(i == 0)
    def _init():
        loss_ref[0, 0] = 0.0

    loss_ref[0, 0] += part

    @pl.when(i == _NBLK - 1)
    def _finish():
        loss_ref[0, 0] = loss_ref[0, 0] * _LOSS_SCALE


def kernel(embeds, centroids, r):
    e2d = embeds.reshape(_B * _N, _D)
    ct = centroids.T
    score_flat, loss, e_out = pl.pallas_call(
        _dist_kernel,
        grid=(_NBLK,),
        in_specs=[
            pl.BlockSpec((_TB, _D), lambda i: (i, 0)),
            pl.BlockSpec((_D, _C), lambda i: (0, 0)),
            pl.BlockSpec(memory_space=pltpu.SMEM),
        ],
        out_specs=[
            pl.BlockSpec((_TB, 1), lambda i: (i, 0)),
            pl.BlockSpec(memory_space=pltpu.SMEM),
            pl.BlockSpec((_TB, _D), lambda i: (i, 0)),
        ],
        out_shape=[
            jax.ShapeDtypeStruct((_B * _N, 1), jnp.float32),
            jax.ShapeDtypeStruct((1, 1), jnp.float32),
            jax.ShapeDtypeStruct((_B * _N, _D), jnp.float32),
        ],
        scratch_shapes=[
            pltpu.VMEM((_D, _C), jnp.bfloat16),
            pltpu.VMEM((1, _C), jnp.float32),
        ],
        compiler_params=pltpu.CompilerParams(
            dimension_semantics=("arbitrary",),
        ),
    )(e2d, ct, r)
    h = 64
    score = score_flat.reshape(_B, 1, h, h)
    return (loss[0, 0], score, e_out.reshape(_B, _N, _D))


# speedup vs baseline: 68.9017x; 1.1535x over previous
import jax
import jax.numpy as jnp
from jax.experimental import pallas as pl
from jax.experimental.pallas import tpu as pltpu

_B = 8
_N = 4096
_D = 512
_C = 1024
_NU = 0.001
_K = 1
_TB = 2048
_NBLK = (_B * _N) // _TB
_LOSS_SCALE = 1.0 / (_NU * _B * _N * _K)


def _dist_kernel(e_ref, ct_ref, r_ref, score_ref, loss_ref, eout_ref,
                 ctm2_ref, cen_ref):
    i = pl.program_id(0)

    @pl.when(i == 0)
    def _prep():
        ct = ct_ref[...]
        ctm2_ref[...] = (-2.0 * ct).astype(jnp.bfloat16)
        cen_ref[...] = jnp.sum(ct * ct, axis=0, keepdims=True)

    e = e_ref[...]
    p = jnp.dot(e.astype(jnp.bfloat16), ctm2_ref[...],
                preferred_element_type=jnp.float32)
    d2 = cen_ref[...] + p
    m = jnp.min(d2, axis=1, keepdims=True)
    feat = jnp.sum(e * e, axis=1, keepdims=True)
    dist = jnp.sqrt(feat + m)
    score_ref[...] = dist
    eout_ref[...] = e
    part = jnp.sum(jnp.maximum(dist - r_ref[0] * r_ref[0], 0.0))

    @pl.when(i == 0)
    def _init():
        loss_ref[0, 0] = 0.0

    loss_ref[0, 0] += part

    @pl.when(i == _NBLK - 1)
    def _finish():
        loss_ref[0, 0] = loss_ref[0, 0] * _LOSS_SCALE


def kernel(embeds, centroids, r):
    e2d = embeds.reshape(_B * _N, _D)
    ct = centroids.T
    score_flat, loss, e_out = pl.pallas_call(
        _dist_kernel,
        grid=(_NBLK,),
        in_specs=[
            pl.BlockSpec((_TB, _D), lambda i: (i, 0)),
            pl.BlockSpec((_D, _C), lambda i: (0, 0)),
            pl.BlockSpec(memory_space=pltpu.SMEM),
        ],
        out_specs=[
            pl.BlockSpec((_TB, 1), lambda i: (i, 0)),
            pl.BlockSpec(memory_space=pltpu.SMEM),
            pl.BlockSpec((_TB, _D), lambda i: (i, 0)),
        ],
        out_shape=[
            jax.ShapeDtypeStruct((_B * _N, 1), jnp.float32),
            jax.ShapeDtypeStruct((1, 1), jnp.float32),
            jax.ShapeDtypeStruct((_B * _N, _D), jnp.float32),
        ],
        scratch_shapes=[
            pltpu.VMEM((_D, _C), jnp.bfloat16),
            pltpu.VMEM((1, _C), jnp.float32),
        ],
        compiler_params=pltpu.CompilerParams(
            dimension_semantics=("arbitrary",),
        ),
    )(e2d, ct, r)
    h = 64
    score = score_flat.reshape(_B, 1, h, h)
    return (loss[0, 0], score, e_out.reshape(_B, _N, _D))


# speedup vs baseline: 73.1686x; 1.0619x over previous
import jax
import jax.numpy as jnp
from jax.experimental import pallas as pl
from jax.experimental.pallas import tpu as pltpu

_B = 8
_N = 4096
_D = 512
_C = 1024
_NU = 0.001
_K = 1
_TB = 4096
_NBLK = (_B * _N) // _TB
_LOSS_SCALE = 1.0 / (_NU * _B * _N * _K)


def _dist_kernel(e_ref, ct_ref, r_ref, score_ref, loss_ref, eout_ref,
                 ctm2_ref, cen_ref):
    i = pl.program_id(0)

    @pl.when(i == 0)
    def _prep():
        ct = ct_ref[...]
        ctm2_ref[...] = (-2.0 * ct).astype(jnp.bfloat16)
        cen_ref[...] = jnp.sum(ct * ct, axis=0, keepdims=True)

    e = e_ref[...]
    p = jnp.dot(e.astype(jnp.bfloat16), ctm2_ref[...],
                preferred_element_type=jnp.float32)
    d2 = cen_ref[...] + p
    m = jnp.min(d2, axis=1, keepdims=True)
    feat = jnp.sum(e * e, axis=1, keepdims=True)
    dist = jnp.sqrt(feat + m)
    score_ref[...] = dist
    eout_ref[...] = e
    part = jnp.sum(jnp.maximum(dist - r_ref[0] * r_ref[0], 0.0))

    @pl.when(i == 0)
    def _init():
        loss_ref[0, 0] = 0.0

    loss_ref[0, 0] += part

    @pl.when(i == _NBLK - 1)
    def _finish():
        loss_ref[0, 0] = loss_ref[0, 0] * _LOSS_SCALE


def kernel(embeds, centroids, r):
    e2d = embeds.reshape(_B * _N, _D)
    ct = centroids.T
    score_flat, loss, e_out = pl.pallas_call(
        _dist_kernel,
        grid=(_NBLK,),
        in_specs=[
            pl.BlockSpec((_TB, _D), lambda i: (i, 0)),
            pl.BlockSpec((_D, _C), lambda i: (0, 0)),
            pl.BlockSpec(memory_space=pltpu.SMEM),
        ],
        out_specs=[
            pl.BlockSpec((_TB, 1), lambda i: (i, 0)),
            pl.BlockSpec(memory_space=pltpu.SMEM),
            pl.BlockSpec((_TB, _D), lambda i: (i, 0)),
        ],
        out_shape=[
            jax.ShapeDtypeStruct((_B * _N, 1), jnp.float32),
            jax.ShapeDtypeStruct((1, 1), jnp.float32),
            jax.ShapeDtypeStruct((_B * _N, _D), jnp.float32),
        ],
        scratch_shapes=[
            pltpu.VMEM((_D, _C), jnp.bfloat16),
            pltpu.VMEM((1, _C), jnp.float32),
        ],
        compiler_params=pltpu.CompilerParams(
            dimension_semantics=("arbitrary",),
        ),
    )(e2d, ct, r)
    h = 64
    score = score_flat.reshape(_B, 1, h, h)
    return (loss[0, 0], score, e_out.reshape(_B, _N, _D))
